# baseline (speedup 1.0000x reference)
import jax
import jax.numpy as jnp
from jax.experimental import pallas as pl
from jax.experimental.pallas import tpu as pltpu

IN_F = 4096
OUT_F = 4096
BATCH = 64
N_BLK = 256


def _mm_kernel(x_ref, w_ref, o_ref):
    o_ref[...] = jax.lax.dot_general(
        w_ref[...], x_ref[...],
        dimension_numbers=(((0,), (0,)), ((), ())),
        preferred_element_type=jnp.float32,
    )


def kernel(in_values, weights):
    grid = (OUT_F // N_BLK,)
    return pl.pallas_call(
        _mm_kernel,
        grid=grid,
        in_specs=[
            pl.BlockSpec((IN_F, BATCH), lambda i: (0, 0)),
            pl.BlockSpec((IN_F, N_BLK), lambda i: (0, i)),
        ],
        out_specs=pl.BlockSpec((N_BLK, BATCH), lambda i: (i, 0)),
        out_shape=jax.ShapeDtypeStruct((OUT_F, BATCH), jnp.float32),
        compiler_params=pltpu.CompilerParams(
            dimension_semantics=("arbitrary",),
        ),
    )(in_values, weights)


# speedup vs baseline: 1.1181x; 1.1181x over previous
import jax
import jax.numpy as jnp
from jax.experimental import pallas as pl
from jax.experimental.pallas import tpu as pltpu

IN_F = 4096
OUT_F = 4096
BATCH = 64
K_BLK = 512


def _mm_kernel(x_ref, w_ref, o_ref):
    acc = jax.lax.dot_general(
        w_ref[...], x_ref[...],
        dimension_numbers=(((0,), (0,)), ((), ())),
        preferred_element_type=jnp.float32,
    )

    @pl.when(pl.program_id(0) == 0)
    def _init():
        o_ref[...] = acc

    @pl.when(pl.program_id(0) != 0)
    def _accum():
        o_ref[...] += acc


def kernel(in_values, weights):
    grid = (IN_F // K_BLK,)
    return pl.pallas_call(
        _mm_kernel,
        grid=grid,
        in_specs=[
            pl.BlockSpec((K_BLK, BATCH), lambda k: (k, 0)),
            pl.BlockSpec((K_BLK, OUT_F), lambda k: (k, 0)),
        ],
        out_specs=pl.BlockSpec((OUT_F, BATCH), lambda k: (0, 0)),
        out_shape=jax.ShapeDtypeStruct((OUT_F, BATCH), jnp.float32),
        compiler_params=pltpu.CompilerParams(
            dimension_semantics=("arbitrary",),
        ),
    )(in_values, weights)
